# Initial kernel scaffold; baseline (speedup 1.0000x reference)
import jax
import jax.numpy as jnp
from jax.experimental import pallas as pl


def kernel(x, edge_index, W_enc, b_enc, Wl, bl, Wr, W_dec, b_dec):
    raise NotImplementedError("write your pallas kernel here")


# speedup vs baseline: 5.5429x; 5.5429x over previous
import functools

import jax
import jax.numpy as jnp
from jax import lax
from jax.experimental import pallas as pl
from jax.experimental.pallas import tpu as pltpu
from jax.experimental.pallas import tpu_sc as plsc

NC = 2
NS = 16
NW = NC * NS
CH = 128
CW = 16


def _zero_divisor(stripe):
  zb = min(stripe, CH)
  while stripe % zb:
    zb -= 1
  return zb


def _stripes(N):
  sp_base = (N // NS) // 8 * 8
  rem = N - NS * sp_base
  assert sp_base > 0 and rem % 8 == 0 and rem <= CH
  return sp_base, rem


def _edge_sums(h, src, dst):
  N, H = h.shape
  E = src.shape[0]
  epw = E // NW
  nfull, tail = divmod(epw, CH)
  sp, rem = _stripes(N)
  zb = _zero_divisor(sp)

  mesh = plsc.VectorSubcoreMesh(core_axis_name="c", subcore_axis_name="s")

  scratch = [
      pltpu.VMEM((CH,), jnp.int32),
      pltpu.VMEM((CH,), jnp.int32),
      pltpu.VMEM((CH, H), jnp.float32),
      pltpu.VMEM_SHARED((N, H), jnp.float32),
      pltpu.SemaphoreType.DMA,
  ]
  if tail:
    scratch += [
        pltpu.VMEM((tail,), jnp.int32),
        pltpu.VMEM((tail,), jnp.int32),
        pltpu.VMEM((tail, H), jnp.float32),
    ]

  @functools.partial(
      pl.kernel,
      out_type=jax.ShapeDtypeStruct((NC, N, H), jnp.float32),
      mesh=mesh,
      scratch_types=scratch,
  )
  def k(h_hbm, src_hbm, dst_hbm, out_hbm, src_v, dst_v, rows_v, acc, sem,
        *tails):
    cid = lax.axis_index("c")
    sid = lax.axis_index("s")

    @pl.loop(0, zb)
    def _(r):
      @pl.loop(0, H, step=16)
      def _(c0):
        rows_v[r, pl.ds(c0, 16)] = jnp.zeros((16,), jnp.float32)

    @pl.loop(0, sp, step=zb)
    def _(r0):
      pltpu.sync_copy(rows_v.at[pl.ds(0, zb)], acc.at[pl.ds(sid * sp + r0, zb)])

    if rem:
      @pl.when(sid == NS - 1)
      def _():
        pltpu.sync_copy(rows_v.at[pl.ds(0, rem)], acc.at[pl.ds(NS * sp, rem)])

    plsc.subcore_barrier()

    ebase = (cid * NS + sid) * epw

    @pl.loop(0, nfull)
    def _(g):
      base = ebase + g * CH
      pltpu.sync_copy(src_hbm.at[pl.ds(base, CH)], src_v)
      pltpu.sync_copy(dst_hbm.at[pl.ds(base, CH)], dst_v)
      pltpu.async_copy(h_hbm.at[src_v], rows_v, sem).wait()
      pltpu.sync_copy(rows_v, acc.at[dst_v], add=True)

    if tail:
      src_t, dst_t, rows_t = tails
      base = ebase + nfull * CH
      pltpu.sync_copy(src_hbm.at[pl.ds(base, tail)], src_t)
      pltpu.sync_copy(dst_hbm.at[pl.ds(base, tail)], dst_t)
      pltpu.async_copy(h_hbm.at[src_t], rows_t, sem).wait()
      pltpu.sync_copy(rows_t, acc.at[dst_t], add=True)

    plsc.subcore_barrier()
    pltpu.sync_copy(acc.at[pl.ds(sid * sp, sp)],
                    out_hbm.at[cid, pl.ds(sid * sp, sp)])
    if rem:
      @pl.when(sid == NS - 1)
      def _():
        pltpu.sync_copy(acc.at[pl.ds(NS * sp, rem)],
                        out_hbm.at[cid, pl.ds(NS * sp, rem)])

  return k(h, src, dst)


def _edge_counts(dst, n_nodes):
  N = n_nodes
  E = dst.shape[0]
  epw = E // NW
  nfull, tail = divmod(epw, CH)
  sp, rem = _stripes(N)
  zb = _zero_divisor(sp)

  mesh = plsc.VectorSubcoreMesh(core_axis_name="c", subcore_axis_name="s")

  scratch = [
      pltpu.VMEM((CH,), jnp.int32),
      pltpu.VMEM((CH, CW), jnp.float32),
      pltpu.VMEM_SHARED((N, CW), jnp.float32),
  ]
  if tail:
    scratch += [pltpu.VMEM((tail,), jnp.int32)]

  @functools.partial(
      pl.kernel,
      out_type=jax.ShapeDtypeStruct((NC, N, CW), jnp.float32),
      mesh=mesh,
      scratch_types=scratch,
  )
  def k(dst_hbm, out_hbm, dst_v, ones_v, acc, *tails):
    cid = lax.axis_index("c")
    sid = lax.axis_index("s")

    @pl.loop(0, CH)
    def _(r):
      ones_v[r, :] = jnp.zeros((CW,), jnp.float32)

    @pl.loop(0, sp, step=zb)
    def _(r0):
      pltpu.sync_copy(ones_v.at[pl.ds(0, zb)], acc.at[pl.ds(sid * sp + r0, zb)])

    if rem:
      @pl.when(sid == NS - 1)
      def _():
        pltpu.sync_copy(ones_v.at[pl.ds(0, rem)], acc.at[pl.ds(NS * sp, rem)])

    @pl.loop(0, CH)
    def _(r):
      ones_v[r, :] = jnp.ones((CW,), jnp.float32)

    plsc.subcore_barrier()

    ebase = (cid * NS + sid) * epw

    @pl.loop(0, nfull)
    def _(g):
      base = ebase + g * CH
      pltpu.sync_copy(dst_hbm.at[pl.ds(base, CH)], dst_v)
      pltpu.sync_copy(ones_v, acc.at[dst_v], add=True)

    if tail:
      (dst_t,) = tails
      base = ebase + nfull * CH
      pltpu.sync_copy(dst_hbm.at[pl.ds(base, tail)], dst_t)
      pltpu.sync_copy(ones_v.at[pl.ds(0, tail)], acc.at[dst_t], add=True)

    plsc.subcore_barrier()
    pltpu.sync_copy(acc.at[pl.ds(sid * sp, sp)],
                    out_hbm.at[cid, pl.ds(sid * sp, sp)])
    if rem:
      @pl.when(sid == NS - 1)
      def _():
        pltpu.sync_copy(acc.at[pl.ds(NS * sp, rem)],
                        out_hbm.at[cid, pl.ds(NS * sp, rem)])

  return k(dst)


_DOT = (((1,), (1,)), ((), ()))


def _encoder(x, W, b):
  M, F = x.shape
  H = W.shape[0]
  bm = 1000

  def body(x_ref, w_ref, b_ref, o_ref):
    o_ref[...] = jnp.maximum(
        lax.dot_general(x_ref[...], w_ref[...], _DOT,
                        preferred_element_type=jnp.float32) + b_ref[...], 0.0)

  return pl.pallas_call(
      body,
      grid=(M // bm,),
      in_specs=[pl.BlockSpec((bm, F), lambda i: (i, 0)),
                pl.BlockSpec((H, F), lambda i: (0, 0)),
                pl.BlockSpec((1, H), lambda i: (0, 0))],
      out_specs=pl.BlockSpec((bm, H), lambda i: (i, 0)),
      out_shape=jax.ShapeDtypeStruct((M, H), jnp.float32),
  )(x, W, b.reshape(1, H))


def _sage_layer(sums, cnts, h, Wl_i, bl_i, Wr_i):
  N, H = h.shape
  bm = 1000

  def body(pa, pb, ca, cb, h_ref, wl, wr, b_ref, o_ref):
    cnt = ca[...][0] + cb[...][0]
    inv = 1.0 / jnp.maximum(cnt[:, 0:1], 1.0)
    mean = (pa[...][0] + pb[...][0]) * inv
    acc = lax.dot_general(mean, wl[...], _DOT,
                          preferred_element_type=jnp.float32)
    acc = acc + lax.dot_general(h_ref[...], wr[...], _DOT,
                                preferred_element_type=jnp.float32)
    o_ref[...] = jnp.maximum(acc + b_ref[...], 0.0)

  return pl.pallas_call(
      body,
      grid=(N // bm,),
      in_specs=[
          pl.BlockSpec((1, bm, H), lambda i: (0, i, 0)),
          pl.BlockSpec((1, bm, H), lambda i: (1, i, 0)),
          pl.BlockSpec((1, bm, CW), lambda i: (0, i, 0)),
          pl.BlockSpec((1, bm, CW), lambda i: (1, i, 0)),
          pl.BlockSpec((bm, H), lambda i: (i, 0)),
          pl.BlockSpec((H, H), lambda i: (0, 0)),
          pl.BlockSpec((H, H), lambda i: (0, 0)),
          pl.BlockSpec((1, H), lambda i: (0, 0)),
      ],
      out_specs=pl.BlockSpec((bm, H), lambda i: (i, 0)),
      out_shape=jax.ShapeDtypeStruct((N, H), jnp.float32),
  )(sums, sums, cnts, cnts, h, Wl_i, Wr_i, bl_i.reshape(1, H))


def _decoder(h, W, b):
  N, H = h.shape
  C = W.shape[0]
  bm = 1000

  def body(h_ref, w_ref, b_ref, o_ref):
    o_ref[...] = lax.dot_general(h_ref[...], w_ref[...], _DOT,
                                 preferred_element_type=jnp.float32) + b_ref[...]

  return pl.pallas_call(
      body,
      grid=(N // bm,),
      in_specs=[pl.BlockSpec((bm, H), lambda i: (i, 0)),
                pl.BlockSpec((C, H), lambda i: (0, 0)),
                pl.BlockSpec((1, C), lambda i: (0, 0))],
      out_specs=pl.BlockSpec((bm, C), lambda i: (i, 0)),
      out_shape=jax.ShapeDtypeStruct((N, C), jnp.float32),
  )(h, W, b.reshape(1, C))


def kernel(x, edge_index, W_enc, b_enc, Wl, bl, Wr, W_dec, b_dec):
  src = edge_index[0]
  dst = edge_index[1]
  n_nodes = x.shape[0]
  n_layers = Wl.shape[0]

  cnts = _edge_counts(dst, n_nodes)
  h = _encoder(x, W_enc, b_enc)
  for i in range(n_layers):
    sums = _edge_sums(h, src, dst)
    h = _sage_layer(sums, cnts, h, Wl[i], bl[i], Wr[i])
  return _decoder(h, W_dec, b_dec)


# speedup vs baseline: 9.5187x; 1.7173x over previous
import functools

import jax
import jax.numpy as jnp
from jax import lax
from jax.experimental import pallas as pl
from jax.experimental.pallas import tpu as pltpu
from jax.experimental.pallas import tpu_sc as plsc

NC = 2
NS = 16
NW = NC * NS
CH = 128
CW = 16


def _stripes(N):
  sp = (N // NS) // 8 * 8
  rem = N - NS * sp
  assert sp > 0 and rem % 8 == 0 and rem <= CH
  return sp, rem


def _zero_divisor(stripe):
  zb = min(stripe, CH)
  while stripe % zb or zb % 8:
    zb -= 1
  return zb


def _make_edge_sums(N, H, nrows, nf, tail, sp, rem, zb):
  npairs = (nf - 2) // 2 if nf % 2 == 0 else (nf - 1) // 2
  mesh = plsc.VectorSubcoreMesh(core_axis_name="c", subcore_axis_name="s")

  scratch = [
      pltpu.VMEM((nrows, CH), jnp.int32),
      pltpu.VMEM((CH, H), jnp.float32),
      pltpu.VMEM((CH, H), jnp.float32),
      pltpu.VMEM((1, CH), jnp.int32),
      pltpu.VMEM((1, CH), jnp.int32),
      pltpu.VMEM((CH,), jnp.int32),
      pltpu.VMEM((CH,), jnp.int32),
      pltpu.VMEM_SHARED((N, H), jnp.float32),
      pltpu.SemaphoreType.DMA,
      pltpu.SemaphoreType.DMA,
      pltpu.SemaphoreType.DMA,
      pltpu.SemaphoreType.DMA,
  ]
  if tail:
    scratch += [
        pltpu.VMEM((tail,), jnp.int32),
        pltpu.VMEM((tail,), jnp.int32),
        pltpu.VMEM((tail, H), jnp.float32),
    ]

  @functools.partial(
      pl.kernel,
      out_type=jax.ShapeDtypeStruct((NC, N, H), jnp.float32),
      mesh=mesh,
      scratch_types=scratch,
  )
  def k(h_hbm, src_hbm, dst_hbm, out_hbm, src_v, buf_a, buf_b, d_a, d_b,
        di_a, di_b, acc, sem_a, sem_b, sem_da, sem_db, *tails):
    cid = lax.axis_index("c")
    sid = lax.axis_index("s")
    wid = cid * NS + sid

    pltpu.sync_copy(src_hbm.at[wid], src_v)

    @pl.loop(0, zb)
    def _(r):
      @pl.loop(0, H, step=16)
      def _(c0):
        buf_a[r, pl.ds(c0, 16)] = jnp.zeros((16,), jnp.float32)

    @pl.loop(0, sp, step=zb)
    def _(r0):
      pltpu.sync_copy(buf_a.at[pl.ds(0, zb)], acc.at[pl.ds(sid * sp + r0, zb)])

    if rem:
      @pl.when(sid == NS - 1)
      def _():
        pltpu.sync_copy(buf_a.at[pl.ds(0, rem)], acc.at[pl.ds(NS * sp, rem)])

    plsc.subcore_barrier()

    def start(g, buf, d, sem_g, sem_d):
      pltpu.async_copy(h_hbm.at[src_v.at[g]], buf, sem_g)
      pltpu.async_copy(dst_hbm.at[wid, g], d, sem_d)

    def finish(g, buf, d, di, sem_g, sem_d):
      pltpu.make_async_copy(h_hbm.at[src_v.at[g]], buf, sem_g).wait()
      pltpu.make_async_copy(dst_hbm.at[wid, g], d, sem_d).wait()
      for t in range(0, CH, 16):
        di[pl.ds(t, 16)] = d[0, pl.ds(t, 16)]
      pltpu.sync_copy(buf, acc.at[di], add=True)

    start(0, buf_a, d_a, sem_a, sem_da)

    @pl.loop(0, npairs)
    def _(p):
      g = 2 * p
      start(g + 1, buf_b, d_b, sem_b, sem_db)
      finish(g, buf_a, d_a, di_a, sem_a, sem_da)
      start(g + 2, buf_a, d_a, sem_a, sem_da)
      finish(g + 1, buf_b, d_b, di_b, sem_b, sem_db)

    if nf % 2 == 0:
      start(nf - 1, buf_b, d_b, sem_b, sem_db)
      finish(nf - 2, buf_a, d_a, di_a, sem_a, sem_da)
      finish(nf - 1, buf_b, d_b, di_b, sem_b, sem_db)
    else:
      finish(nf - 1, buf_a, d_a, di_a, sem_a, sem_da)

    if tail:
      st, dt, rows_t = tails
      pltpu.sync_copy(dst_hbm.at[wid, nf], d_a)
      for t in range(0, tail, 16):
        st[pl.ds(t, 16)] = src_v[nf, pl.ds(t, 16)]
        dt[pl.ds(t, 16)] = d_a[0, pl.ds(t, 16)]
      pltpu.async_copy(h_hbm.at[st], rows_t, sem_a).wait()
      pltpu.sync_copy(rows_t, acc.at[dt], add=True)

    plsc.subcore_barrier()
    pltpu.sync_copy(acc.at[pl.ds(sid * sp, sp)],
                    out_hbm.at[cid, pl.ds(sid * sp, sp)])
    if rem:
      @pl.when(sid == NS - 1)
      def _():
        pltpu.sync_copy(acc.at[pl.ds(NS * sp, rem)],
                        out_hbm.at[cid, pl.ds(NS * sp, rem)])

  return k


def _edge_counts(dst4, ones_hbm, zeros_hbm, N, H, nf, tail):
  sp, rem = _stripes(N)
  zb = _zero_divisor(sp)

  mesh = plsc.VectorSubcoreMesh(core_axis_name="c", subcore_axis_name="s")

  scratch = [
      pltpu.VMEM((1, CH), jnp.int32),
      pltpu.VMEM((CH, H), jnp.float32),
      pltpu.VMEM((zb, H), jnp.float32),
      pltpu.VMEM((CH,), jnp.int32),
      pltpu.VMEM_SHARED((N, H), jnp.float32),
  ]
  if tail:
    scratch += [pltpu.VMEM((tail,), jnp.int32)]

  @functools.partial(
      pl.kernel,
      out_type=jax.ShapeDtypeStruct((NC, N, H), jnp.float32),
      mesh=mesh,
      scratch_types=scratch,
  )
  def k(dst_hbm, ones_h, zeros_h, out_hbm, d, ones_v, zero_v, di, acc, *tails):
    cid = lax.axis_index("c")
    sid = lax.axis_index("s")
    wid = cid * NS + sid

    pltpu.sync_copy(ones_h, ones_v)
    pltpu.sync_copy(zeros_h, zero_v)

    @pl.loop(0, sp, step=zb)
    def _(r0):
      pltpu.sync_copy(zero_v, acc.at[pl.ds(sid * sp + r0, zb)])

    if rem:
      @pl.when(sid == NS - 1)
      def _():
        pltpu.sync_copy(zero_v.at[pl.ds(0, rem)], acc.at[pl.ds(NS * sp, rem)])

    plsc.subcore_barrier()

    @pl.loop(0, nf)
    def _(g):
      pltpu.sync_copy(dst_hbm.at[wid, g], d)
      for t in range(0, CH, 16):
        di[pl.ds(t, 16)] = d[0, pl.ds(t, 16)]
      pltpu.sync_copy(ones_v, acc.at[di], add=True)

    if tail:
      (dt,) = tails
      pltpu.sync_copy(dst_hbm.at[wid, nf], d)
      for t in range(0, tail, 16):
        dt[pl.ds(t, 16)] = d[0, pl.ds(t, 16)]
      pltpu.sync_copy(ones_v.at[pl.ds(0, tail)], acc.at[dt], add=True)

    plsc.subcore_barrier()
    pltpu.sync_copy(acc.at[pl.ds(sid * sp, sp)],
                    out_hbm.at[cid, pl.ds(sid * sp, sp)])
    if rem:
      @pl.when(sid == NS - 1)
      def _():
        pltpu.sync_copy(acc.at[pl.ds(NS * sp, rem)],
                        out_hbm.at[cid, pl.ds(NS * sp, rem)])

  return k(dst4, ones_hbm, zeros_hbm)


_DOT = (((1,), (1,)), ((), ()))


def _encoder(x, W, b):
  M, F = x.shape
  H = W.shape[0]
  bm = 1000

  def body(x_ref, w_ref, b_ref, o_ref):
    o_ref[...] = jnp.maximum(
        lax.dot_general(x_ref[...], w_ref[...], _DOT,
                        preferred_element_type=jnp.float32) + b_ref[...], 0.0)

  return pl.pallas_call(
      body,
      grid=(M // bm,),
      in_specs=[pl.BlockSpec((bm, F), lambda i: (i, 0)),
                pl.BlockSpec((H, F), lambda i: (0, 0)),
                pl.BlockSpec((1, H), lambda i: (0, 0))],
      out_specs=pl.BlockSpec((bm, H), lambda i: (i, 0)),
      out_shape=jax.ShapeDtypeStruct((M, H), jnp.float32),
  )(x, W, b.reshape(1, H))


def _sage_layer(sums, cnts, h, Wl_i, bl_i, Wr_i):
  N, H = h.shape
  bm = 1000

  def body(pa, pb, ca, cb, h_ref, wl, wr, b_ref, o_ref):
    cnt = ca[...][0] + cb[...][0]
    inv = 1.0 / jnp.maximum(cnt[:, 0:1], 1.0)
    mean = (pa[...][0] + pb[...][0]) * inv
    acc = lax.dot_general(mean, wl[...], _DOT,
                          preferred_element_type=jnp.float32)
    acc = acc + lax.dot_general(h_ref[...], wr[...], _DOT,
                                preferred_element_type=jnp.float32)
    o_ref[...] = jnp.maximum(acc + b_ref[...], 0.0)

  return pl.pallas_call(
      body,
      grid=(N // bm,),
      in_specs=[
          pl.BlockSpec((1, bm, H), lambda i: (0, i, 0)),
          pl.BlockSpec((1, bm, H), lambda i: (1, i, 0)),
          pl.BlockSpec((1, bm, H), lambda i: (0, i, 0)),
          pl.BlockSpec((1, bm, H), lambda i: (1, i, 0)),
          pl.BlockSpec((bm, H), lambda i: (i, 0)),
          pl.BlockSpec((H, H), lambda i: (0, 0)),
          pl.BlockSpec((H, H), lambda i: (0, 0)),
          pl.BlockSpec((1, H), lambda i: (0, 0)),
      ],
      out_specs=pl.BlockSpec((bm, H), lambda i: (i, 0)),
      out_shape=jax.ShapeDtypeStruct((N, H), jnp.float32),
  )(sums, sums, cnts, cnts, h, Wl_i, Wr_i, bl_i.reshape(1, H))


def _decoder(h, W, b):
  N, H = h.shape
  C = W.shape[0]
  bm = 1000

  def body(h_ref, w_ref, b_ref, o_ref):
    o_ref[...] = lax.dot_general(h_ref[...], w_ref[...], _DOT,
                                 preferred_element_type=jnp.float32) + b_ref[...]

  return pl.pallas_call(
      body,
      grid=(N // bm,),
      in_specs=[pl.BlockSpec((bm, H), lambda i: (i, 0)),
                pl.BlockSpec((C, H), lambda i: (0, 0)),
                pl.BlockSpec((1, C), lambda i: (0, 0))],
      out_specs=pl.BlockSpec((bm, C), lambda i: (i, 0)),
      out_shape=jax.ShapeDtypeStruct((N, C), jnp.float32),
  )(h, W, b.reshape(1, C))


def kernel(x, edge_index, W_enc, b_enc, Wl, bl, Wr, W_dec, b_dec):
  E = edge_index.shape[1]
  N, H = x.shape[0], W_enc.shape[0]
  assert E % NW == 0
  epw = E // NW
  nf, tail = divmod(epw, CH)
  assert tail % 16 == 0
  nrows = nf + (1 if tail else 0)
  ei = edge_index.reshape(2, NW, epw)
  padn = nrows * CH - epw
  if padn:
    ei = jnp.pad(ei, ((0, 0), (0, 0), (0, padn)))
  ei = ei.reshape(2, NW, nrows, CH)
  src3 = ei[0]
  dst4 = ei[1].reshape(NW, nrows, 1, CH)

  sp, rem = _stripes(N)
  zb = _zero_divisor(sp)
  sums_k = _make_edge_sums(N, H, nrows, nf, tail, sp, rem, zb)

  ones_c = jnp.ones((CH, H), jnp.float32)
  zeros_c = jnp.zeros((_zero_divisor(_stripes(N)[0]), H), jnp.float32)
  cnts = _edge_counts(dst4, ones_c, zeros_c, N, H, nf, tail)
  h = _encoder(x, W_enc, b_enc)
  for i in range(Wl.shape[0]):
    sums = sums_k(h, src3, dst4)
    h = _sage_layer(sums, cnts, h, Wl[i], bl[i], Wr[i])
  return _decoder(h, W_dec, b_dec)


# speedup vs baseline: 10.9172x; 1.1469x over previous
import dataclasses
import functools

import jax
import jax.numpy as jnp
from jax import lax
from jax.experimental import pallas as pl
from jax.experimental.pallas import tpu as pltpu
from jax.experimental.pallas import tpu_sc as plsc

NC = 2
NS = 16
NW = NC * NS
CH = 128
CW = 16


def _stripes(N):
  sp = (N // NS) // 8 * 8
  rem = N - NS * sp
  assert sp > 0 and rem % 8 == 0 and rem <= CH
  return sp, rem


def _zero_divisor(stripe):
  zb = min(stripe, CH)
  while stripe % zb or zb % 8:
    zb -= 1
  return zb


def _make_edge_sums(N, H, nrows, nf, tail, sp, rem, zb):
  npairs = (nf - 2) // 2 if nf % 2 == 0 else (nf - 1) // 2
  mesh = plsc.VectorSubcoreMesh(core_axis_name="c", subcore_axis_name="s")

  scratch = [
      pltpu.VMEM((nrows, CH), jnp.int32),
      pltpu.VMEM((CH, H), jnp.float32),
      pltpu.VMEM((CH, H), jnp.float32),
      pltpu.VMEM((1, CH), jnp.int32),
      pltpu.VMEM((1, CH), jnp.int32),
      pltpu.VMEM((CH,), jnp.int32),
      pltpu.VMEM((CH,), jnp.int32),
      pltpu.VMEM_SHARED((N, H), jnp.float32),
      pltpu.SemaphoreType.DMA,
      pltpu.SemaphoreType.DMA,
      pltpu.SemaphoreType.DMA,
      pltpu.SemaphoreType.DMA,
  ]
  if tail:
    scratch += [
        pltpu.VMEM((tail,), jnp.int32),
        pltpu.VMEM((tail,), jnp.int32),
        pltpu.VMEM((tail, H), jnp.float32),
    ]

  @functools.partial(
      pl.kernel,
      out_type=jax.ShapeDtypeStruct((NC, N, H), jnp.float32),
      mesh=mesh,
      scratch_types=scratch,
  )
  def k(h_hbm, src_hbm, dst_hbm, out_hbm, src_v, buf_a, buf_b, d_a, d_b,
        di_a, di_b, acc, sem_a, sem_b, sem_da, sem_db, *tails):
    cid = lax.axis_index("c")
    sid = lax.axis_index("s")
    wid = cid * NS + sid

    pltpu.sync_copy(src_hbm.at[wid], src_v)

    @pl.loop(0, zb)
    def _(r):
      @pl.loop(0, H, step=16)
      def _(c0):
        buf_a[r, pl.ds(c0, 16)] = jnp.zeros((16,), jnp.float32)

    @pl.loop(0, sp, step=zb)
    def _(r0):
      pltpu.sync_copy(buf_a.at[pl.ds(0, zb)], acc.at[pl.ds(sid * sp + r0, zb)])

    if rem:
      @pl.when(sid == NS - 1)
      def _():
        pltpu.sync_copy(buf_a.at[pl.ds(0, rem)], acc.at[pl.ds(NS * sp, rem)])

    plsc.subcore_barrier()

    def start(g, buf, d, sem_g, sem_d):
      pltpu.async_copy(h_hbm.at[src_v.at[g]], buf, sem_g)
      pltpu.async_copy(dst_hbm.at[wid, g], d, sem_d)

    def finish(g, buf, d, di, sem_g, sem_d):
      pltpu.make_async_copy(h_hbm.at[src_v.at[g]], buf, sem_g).wait()
      pltpu.make_async_copy(dst_hbm.at[wid, g], d, sem_d).wait()
      for t in range(0, CH, 16):
        di[pl.ds(t, 16)] = d[0, pl.ds(t, 16)]
      pltpu.sync_copy(buf, acc.at[di], add=True)

    start(0, buf_a, d_a, sem_a, sem_da)

    @pl.loop(0, npairs)
    def _(p):
      g = 2 * p
      start(g + 1, buf_b, d_b, sem_b, sem_db)
      finish(g, buf_a, d_a, di_a, sem_a, sem_da)
      start(g + 2, buf_a, d_a, sem_a, sem_da)
      finish(g + 1, buf_b, d_b, di_b, sem_b, sem_db)

    if nf % 2 == 0:
      start(nf - 1, buf_b, d_b, sem_b, sem_db)
      finish(nf - 2, buf_a, d_a, di_a, sem_a, sem_da)
      finish(nf - 1, buf_b, d_b, di_b, sem_b, sem_db)
    else:
      finish(nf - 1, buf_a, d_a, di_a, sem_a, sem_da)

    if tail:
      st, dt, rows_t = tails
      pltpu.sync_copy(dst_hbm.at[wid, nf], d_a)
      for t in range(0, tail, 16):
        st[pl.ds(t, 16)] = src_v[nf, pl.ds(t, 16)]
        dt[pl.ds(t, 16)] = d_a[0, pl.ds(t, 16)]
      pltpu.async_copy(h_hbm.at[st], rows_t, sem_a).wait()
      pltpu.sync_copy(rows_t, acc.at[dt], add=True)

    plsc.subcore_barrier()
    pltpu.sync_copy(acc.at[pl.ds(sid * sp, sp)],
                    out_hbm.at[cid, pl.ds(sid * sp, sp)])
    if rem:
      @pl.when(sid == NS - 1)
      def _():
        pltpu.sync_copy(acc.at[pl.ds(NS * sp, rem)],
                        out_hbm.at[cid, pl.ds(NS * sp, rem)])

  return k


def _edge_counts(dst_flat, N, epw, nf, tail):
  cp = pltpu.CompilerParams()
  if "needs_layout_passes" in pltpu.CompilerParams.__dataclass_fields__:
    cp = dataclasses.replace(cp, needs_layout_passes=False)

  mesh = plsc.VectorSubcoreMesh(core_axis_name="c", subcore_axis_name="s")

  scratch = [
      pltpu.VMEM((CH,), jnp.int32),
      pltpu.VMEM((N,), jnp.float32),
  ]
  if tail:
    scratch += [pltpu.VMEM((tail,), jnp.int32)]

  @functools.partial(
      pl.kernel,
      out_type=jax.ShapeDtypeStruct((NW * N,), jnp.float32),
      mesh=mesh,
      scratch_types=scratch,
      compiler_params=cp,
  )
  def k(dst_hbm, out_hbm, d, hist, *tails):
    cid = lax.axis_index("c")
    sid = lax.axis_index("s")
    wid = cid * NS + sid
    ebase = wid * epw

    @pl.loop(0, N, step=16)
    def _(t):
      hist[pl.ds(t, 16)] = jnp.zeros((16,), jnp.float32)

    ones16 = jnp.ones((16,), jnp.float32)

    @pl.loop(0, nf)
    def _(g):
      pltpu.sync_copy(dst_hbm.at[pl.ds(ebase + g * CH, CH)], d)
      for t in range(0, CH, 16):
        plsc.addupdate_scatter(hist, [d[pl.ds(t, 16)]], ones16)

    if tail:
      (dt,) = tails
      pltpu.sync_copy(dst_hbm.at[pl.ds(ebase + nf * CH, tail)], dt)
      for t in range(0, tail, 16):
        plsc.addupdate_scatter(hist, [dt[pl.ds(t, 16)]], ones16)

    pltpu.sync_copy(hist, out_hbm.at[pl.ds(wid * N, N)])

  return k(dst_flat)


_DOT = (((1,), (1,)), ((), ()))


def _encoder(x, W, b):
  M, F = x.shape
  H = W.shape[0]
  bm = 1000

  def body(x_ref, w_ref, b_ref, o_ref):
    o_ref[...] = jnp.maximum(
        lax.dot_general(x_ref[...], w_ref[...], _DOT,
                        preferred_element_type=jnp.float32) + b_ref[...], 0.0)

  return pl.pallas_call(
      body,
      grid=(M // bm,),
      in_specs=[pl.BlockSpec((bm, F), lambda i: (i, 0)),
                pl.BlockSpec((H, F), lambda i: (0, 0)),
                pl.BlockSpec((1, H), lambda i: (0, 0))],
      out_specs=pl.BlockSpec((bm, H), lambda i: (i, 0)),
      out_shape=jax.ShapeDtypeStruct((M, H), jnp.float32),
  )(x, W, b.reshape(1, H))


def _sage_layer(sums, cnts, h, Wl_i, bl_i, Wr_i):
  N, H = h.shape
  bm = 1000

  def body(pa, pb, c_ref, h_ref, wl, wr, b_ref, o_ref):
    cnt = jnp.sum(c_ref[...], axis=1, keepdims=True)
    inv = 1.0 / jnp.maximum(cnt, 1.0)
    mean = (pa[...][0] + pb[...][0]) * inv
    acc = lax.dot_general(mean, wl[...], _DOT,
                          preferred_element_type=jnp.float32)
    acc = acc + lax.dot_general(h_ref[...], wr[...], _DOT,
                                preferred_element_type=jnp.float32)
    o_ref[...] = jnp.maximum(acc + b_ref[...], 0.0)

  return pl.pallas_call(
      body,
      grid=(N // bm,),
      in_specs=[
          pl.BlockSpec((1, bm, H), lambda i: (0, i, 0)),
          pl.BlockSpec((1, bm, H), lambda i: (1, i, 0)),
          pl.BlockSpec((bm, NW), lambda i: (i, 0)),
          pl.BlockSpec((bm, H), lambda i: (i, 0)),
          pl.BlockSpec((H, H), lambda i: (0, 0)),
          pl.BlockSpec((H, H), lambda i: (0, 0)),
          pl.BlockSpec((1, H), lambda i: (0, 0)),
      ],
      out_specs=pl.BlockSpec((bm, H), lambda i: (i, 0)),
      out_shape=jax.ShapeDtypeStruct((N, H), jnp.float32),
  )(sums, sums, cnts, h, Wl_i, Wr_i, bl_i.reshape(1, H))


def _decoder(h, W, b):
  N, H = h.shape
  C = W.shape[0]
  bm = 1000

  def body(h_ref, w_ref, b_ref, o_ref):
    o_ref[...] = lax.dot_general(h_ref[...], w_ref[...], _DOT,
                                 preferred_element_type=jnp.float32) + b_ref[...]

  return pl.pallas_call(
      body,
      grid=(N // bm,),
      in_specs=[pl.BlockSpec((bm, H), lambda i: (i, 0)),
                pl.BlockSpec((C, H), lambda i: (0, 0)),
                pl.BlockSpec((1, C), lambda i: (0, 0))],
      out_specs=pl.BlockSpec((bm, C), lambda i: (i, 0)),
      out_shape=jax.ShapeDtypeStruct((N, C), jnp.float32),
  )(h, W, b.reshape(1, C))


def kernel(x, edge_index, W_enc, b_enc, Wl, bl, Wr, W_dec, b_dec):
  E = edge_index.shape[1]
  N, H = x.shape[0], W_enc.shape[0]
  assert E % NW == 0
  epw = E // NW
  nf, tail = divmod(epw, CH)
  assert tail % 16 == 0
  nrows = nf + (1 if tail else 0)
  ei = edge_index.reshape(2, NW, epw)
  padn = nrows * CH - epw
  if padn:
    ei = jnp.pad(ei, ((0, 0), (0, 0), (0, padn)))
  ei = ei.reshape(2, NW, nrows, CH)
  src3 = ei[0]
  dst4 = ei[1].reshape(NW, nrows, 1, CH)

  sp, rem = _stripes(N)
  zb = _zero_divisor(sp)
  sums_k = _make_edge_sums(N, H, nrows, nf, tail, sp, rem, zb)

  cnt_flat = _edge_counts(edge_index[1], N, epw, nf, tail)
  cnts = cnt_flat.reshape(NW, N).T
  h = _encoder(x, W_enc, b_enc)
  for i in range(Wl.shape[0]):
    sums = sums_k(h, src3, dst4)
    h = _sage_layer(sums, cnts, h, Wl[i], bl[i], Wr[i])
  return _decoder(h, W_dec, b_dec)


# speedup vs baseline: 11.1486x; 1.0212x over previous
import dataclasses
import functools

import jax
import jax.numpy as jnp
from jax import lax
from jax.experimental import pallas as pl
from jax.experimental.pallas import tpu as pltpu
from jax.experimental.pallas import tpu_sc as plsc

NC = 2
NS = 16
NW = NC * NS
CH = 128
CW = 16


def _stripes(N):
  sp = (N // NS) // 8 * 8
  rem = N - NS * sp
  assert sp > 0 and rem % 8 == 0 and rem <= CH
  return sp, rem


def _zero_divisor(stripe):
  zb = min(stripe, CH)
  while stripe % zb or zb % 8:
    zb -= 1
  return zb


def _make_edge_sums(N, H, nrows, nf, tail, sp, rem, zb):
  npairs = (nf - 2) // 2 if nf % 2 == 0 else (nf - 1) // 2
  mesh = plsc.VectorSubcoreMesh(core_axis_name="c", subcore_axis_name="s")

  scratch = [
      pltpu.VMEM((nrows, CH), jnp.int32),
      pltpu.VMEM((CH, H), jnp.float32),
      pltpu.VMEM((CH, H), jnp.float32),
      pltpu.VMEM((1, CH), jnp.int32),
      pltpu.VMEM((1, CH), jnp.int32),
      pltpu.VMEM((CH,), jnp.int32),
      pltpu.VMEM((CH,), jnp.int32),
      pltpu.VMEM_SHARED((N, H), jnp.float32),
      pltpu.SemaphoreType.DMA,
      pltpu.SemaphoreType.DMA,
      pltpu.SemaphoreType.DMA,
      pltpu.SemaphoreType.DMA,
  ]
  if tail:
    scratch += [
        pltpu.VMEM((tail,), jnp.int32),
        pltpu.VMEM((tail,), jnp.int32),
        pltpu.VMEM((tail, H), jnp.float32),
    ]

  @functools.partial(
      pl.kernel,
      out_type=jax.ShapeDtypeStruct((NC, N, H), jnp.float32),
      mesh=mesh,
      scratch_types=scratch,
  )
  def k(h_hbm, src_hbm, dst_hbm, out_hbm, src_v, buf_a, buf_b, d_a, d_b,
        di_a, di_b, acc, sem_a, sem_b, sem_da, sem_db, *tails):
    cid = lax.axis_index("c")
    sid = lax.axis_index("s")
    wid = cid * NS + sid

    pltpu.sync_copy(src_hbm.at[wid], src_v)

    @pl.loop(0, zb)
    def _(r):
      @pl.loop(0, H, step=16)
      def _(c0):
        buf_a[r, pl.ds(c0, 16)] = jnp.zeros((16,), jnp.float32)

    @pl.loop(0, sp, step=zb)
    def _(r0):
      pltpu.sync_copy(buf_a.at[pl.ds(0, zb)], acc.at[pl.ds(sid * sp + r0, zb)])

    if rem:
      @pl.when(sid == NS - 1)
      def _():
        pltpu.sync_copy(buf_a.at[pl.ds(0, rem)], acc.at[pl.ds(NS * sp, rem)])

    plsc.subcore_barrier()

    def start(g, buf, d, sem_g, sem_d):
      pltpu.async_copy(h_hbm.at[src_v.at[g]], buf, sem_g)
      pltpu.async_copy(dst_hbm.at[wid, g], d, sem_d)

    def finish(g, buf, d, di, sem_g, sem_d):
      pltpu.make_async_copy(h_hbm.at[src_v.at[g]], buf, sem_g).wait()
      pltpu.make_async_copy(dst_hbm.at[wid, g], d, sem_d).wait()
      for t in range(0, CH, 16):
        di[pl.ds(t, 16)] = d[0, pl.ds(t, 16)]
      pltpu.sync_copy(buf, acc.at[di], add=True)

    start(0, buf_a, d_a, sem_a, sem_da)

    @pl.loop(0, npairs)
    def _(p):
      g = 2 * p
      start(g + 1, buf_b, d_b, sem_b, sem_db)
      finish(g, buf_a, d_a, di_a, sem_a, sem_da)
      start(g + 2, buf_a, d_a, sem_a, sem_da)
      finish(g + 1, buf_b, d_b, di_b, sem_b, sem_db)

    if nf % 2 == 0:
      start(nf - 1, buf_b, d_b, sem_b, sem_db)
      finish(nf - 2, buf_a, d_a, di_a, sem_a, sem_da)
      finish(nf - 1, buf_b, d_b, di_b, sem_b, sem_db)
    else:
      finish(nf - 1, buf_a, d_a, di_a, sem_a, sem_da)

    if tail:
      st, dt, rows_t = tails
      pltpu.sync_copy(dst_hbm.at[wid, nf], d_a)
      for t in range(0, tail, 16):
        st[pl.ds(t, 16)] = src_v[nf, pl.ds(t, 16)]
        dt[pl.ds(t, 16)] = d_a[0, pl.ds(t, 16)]
      pltpu.async_copy(h_hbm.at[st], rows_t, sem_a).wait()
      pltpu.sync_copy(rows_t, acc.at[dt], add=True)

    plsc.subcore_barrier()
    pltpu.sync_copy(acc.at[pl.ds(sid * sp, sp)],
                    out_hbm.at[cid, pl.ds(sid * sp, sp)])
    if rem:
      @pl.when(sid == NS - 1)
      def _():
        pltpu.sync_copy(acc.at[pl.ds(NS * sp, rem)],
                        out_hbm.at[cid, pl.ds(NS * sp, rem)])

  return k


def _edge_counts(dst_flat, N, epw, nf, tail):
  cp = pltpu.CompilerParams()
  if "needs_layout_passes" in pltpu.CompilerParams.__dataclass_fields__:
    cp = dataclasses.replace(cp, needs_layout_passes=False)

  mesh = plsc.VectorSubcoreMesh(core_axis_name="c", subcore_axis_name="s")

  scratch = [
      pltpu.VMEM((CH,), jnp.int32),
      pltpu.VMEM((N,), jnp.float32),
  ]
  if tail:
    scratch += [pltpu.VMEM((tail,), jnp.int32)]

  @functools.partial(
      pl.kernel,
      out_type=jax.ShapeDtypeStruct((NW * N,), jnp.float32),
      mesh=mesh,
      scratch_types=scratch,
      compiler_params=cp,
  )
  def k(dst_hbm, out_hbm, d, hist, *tails):
    cid = lax.axis_index("c")
    sid = lax.axis_index("s")
    wid = cid * NS + sid
    ebase = wid * epw

    @pl.loop(0, N, step=16)
    def _(t):
      hist[pl.ds(t, 16)] = jnp.zeros((16,), jnp.float32)

    ones16 = jnp.ones((16,), jnp.float32)

    @pl.loop(0, nf)
    def _(g):
      pltpu.sync_copy(dst_hbm.at[pl.ds(ebase + g * CH, CH)], d)
      for t in range(0, CH, 16):
        plsc.addupdate_scatter(hist, [d[pl.ds(t, 16)]], ones16)

    if tail:
      (dt,) = tails
      pltpu.sync_copy(dst_hbm.at[pl.ds(ebase + nf * CH, tail)], dt)
      for t in range(0, tail, 16):
        plsc.addupdate_scatter(hist, [dt[pl.ds(t, 16)]], ones16)

    pltpu.sync_copy(hist, out_hbm.at[pl.ds(wid * N, N)])

  return k(dst_flat)


_DOT = (((1,), (1,)), ((), ()))


def _encoder(x, W, b):
  M, F = x.shape
  H = W.shape[0]
  bm = 1000

  def body(x_ref, w_ref, b_ref, o_ref):
    o_ref[...] = jnp.maximum(
        lax.dot_general(x_ref[...], w_ref[...], _DOT,
                        preferred_element_type=jnp.float32) + b_ref[...], 0.0)

  return pl.pallas_call(
      body,
      grid=(M // bm,),
      in_specs=[pl.BlockSpec((bm, F), lambda i: (i, 0)),
                pl.BlockSpec((H, F), lambda i: (0, 0)),
                pl.BlockSpec((1, H), lambda i: (0, 0))],
      out_specs=pl.BlockSpec((bm, H), lambda i: (i, 0)),
      out_shape=jax.ShapeDtypeStruct((M, H), jnp.float32),
  )(x, W, b.reshape(1, H))


def _sage_layer(sums, cnts, h, Wl_i, bl_i, Wr_i):
  N, H = h.shape
  bm = 1000

  def body(pa, pb, c_ref, h_ref, wl, wr, b_ref, o_ref):
    cnt = jnp.sum(c_ref[...], axis=1, keepdims=True)
    inv = 1.0 / jnp.maximum(cnt, 1.0)
    mean = (pa[...][0] + pb[...][0]) * inv
    acc = lax.dot_general(mean, wl[...], _DOT,
                          preferred_element_type=jnp.float32)
    acc = acc + lax.dot_general(h_ref[...], wr[...], _DOT,
                                preferred_element_type=jnp.float32)
    o_ref[...] = jnp.maximum(acc + b_ref[...], 0.0)

  return pl.pallas_call(
      body,
      grid=(N // bm,),
      in_specs=[
          pl.BlockSpec((1, bm, H), lambda i: (0, i, 0)),
          pl.BlockSpec((1, bm, H), lambda i: (1, i, 0)),
          pl.BlockSpec((bm, NW), lambda i: (i, 0)),
          pl.BlockSpec((bm, H), lambda i: (i, 0)),
          pl.BlockSpec((H, H), lambda i: (0, 0)),
          pl.BlockSpec((H, H), lambda i: (0, 0)),
          pl.BlockSpec((1, H), lambda i: (0, 0)),
      ],
      out_specs=pl.BlockSpec((bm, H), lambda i: (i, 0)),
      out_shape=jax.ShapeDtypeStruct((N, H), jnp.float32),
  )(sums, sums, cnts, h, Wl_i, Wr_i, bl_i.reshape(1, H))


def _sage_layer_dec(sums, cnts, h, Wl_i, bl_i, Wr_i, W_dec, b_dec):
  N, H = h.shape
  C = W_dec.shape[0]
  bm = 1000

  def body(pa, pb, c_ref, h_ref, wl, wr, b_ref, wd, bd, o_ref):
    cnt = jnp.sum(c_ref[...], axis=1, keepdims=True)
    inv = 1.0 / jnp.maximum(cnt, 1.0)
    mean = (pa[...][0] + pb[...][0]) * inv
    acc = lax.dot_general(mean, wl[...], _DOT,
                          preferred_element_type=jnp.float32)
    acc = acc + lax.dot_general(h_ref[...], wr[...], _DOT,
                                preferred_element_type=jnp.float32)
    hn = jnp.maximum(acc + b_ref[...], 0.0)
    o_ref[...] = lax.dot_general(hn, wd[...], _DOT,
                                 preferred_element_type=jnp.float32) + bd[...]

  return pl.pallas_call(
      body,
      grid=(N // bm,),
      in_specs=[
          pl.BlockSpec((1, bm, H), lambda i: (0, i, 0)),
          pl.BlockSpec((1, bm, H), lambda i: (1, i, 0)),
          pl.BlockSpec((bm, NW), lambda i: (i, 0)),
          pl.BlockSpec((bm, H), lambda i: (i, 0)),
          pl.BlockSpec((H, H), lambda i: (0, 0)),
          pl.BlockSpec((H, H), lambda i: (0, 0)),
          pl.BlockSpec((1, H), lambda i: (0, 0)),
          pl.BlockSpec((C, H), lambda i: (0, 0)),
          pl.BlockSpec((1, C), lambda i: (0, 0)),
      ],
      out_specs=pl.BlockSpec((bm, C), lambda i: (i, 0)),
      out_shape=jax.ShapeDtypeStruct((N, C), jnp.float32),
  )(sums, sums, cnts, h, Wl_i, Wr_i, bl_i.reshape(1, H),
    W_dec, b_dec.reshape(1, C))


def _decoder(h, W, b):
  N, H = h.shape
  C = W.shape[0]
  bm = 1000

  def body(h_ref, w_ref, b_ref, o_ref):
    o_ref[...] = lax.dot_general(h_ref[...], w_ref[...], _DOT,
                                 preferred_element_type=jnp.float32) + b_ref[...]

  return pl.pallas_call(
      body,
      grid=(N // bm,),
      in_specs=[pl.BlockSpec((bm, H), lambda i: (i, 0)),
                pl.BlockSpec((C, H), lambda i: (0, 0)),
                pl.BlockSpec((1, C), lambda i: (0, 0))],
      out_specs=pl.BlockSpec((bm, C), lambda i: (i, 0)),
      out_shape=jax.ShapeDtypeStruct((N, C), jnp.float32),
  )(h, W, b.reshape(1, C))


def kernel(x, edge_index, W_enc, b_enc, Wl, bl, Wr, W_dec, b_dec):
  E = edge_index.shape[1]
  N, H = x.shape[0], W_enc.shape[0]
  assert E % NW == 0
  epw = E // NW
  nf, tail = divmod(epw, CH)
  assert tail % 16 == 0
  nrows = nf + (1 if tail else 0)
  ei = edge_index.reshape(2, NW, epw)
  padn = nrows * CH - epw
  if padn:
    ei = jnp.pad(ei, ((0, 0), (0, 0), (0, padn)))
  ei = ei.reshape(2, NW, nrows, CH)
  src3 = ei[0]
  dst4 = ei[1].reshape(NW, nrows, 1, CH)

  sp, rem = _stripes(N)
  zb = _zero_divisor(sp)
  sums_k = _make_edge_sums(N, H, nrows, nf, tail, sp, rem, zb)

  cnt_flat = _edge_counts(edge_index[1], N, epw, nf, tail)
  cnts = cnt_flat.reshape(NW, N).T
  h = _encoder(x, W_enc, b_enc)
  L = Wl.shape[0]
  for i in range(L - 1):
    sums = sums_k(h, src3, dst4)
    h = _sage_layer(sums, cnts, h, Wl[i], bl[i], Wr[i])
  sums = sums_k(h, src3, dst4)
  return _sage_layer_dec(sums, cnts, h, Wl[L - 1], bl[L - 1], Wr[L - 1],
                         W_dec, b_dec)
